# baseline (speedup 1.0000x reference)
import jax
import jax.numpy as jnp
from jax.experimental import pallas as pl
from jax.experimental.pallas import tpu as pltpu

N = 100000
D = 128
BN = 10000
BC = 2500
NCHUNK = BN // BC


def _fused_gate_kernel(x1_ref, x2_ref, w1a_ref, w1b_ref, b1_ref,
                       w2a_ref, w2b_ref, b2_ref, out_ref,
                       out_buf, out_sem):
    i = pl.program_id(0)
    nsteps = pl.num_programs(0)
    buf = jax.lax.rem(i, 2)
    w1a = w1a_ref[...]
    w1b = w1b_ref[...]
    w2a = w2a_ref[...]
    w2b = w2b_ref[...]
    b1 = b1_ref[...]
    b2 = b2_ref[...]
    for c in range(NCHUNK):
        @pl.when(i >= 2)
        def _wait_prev():
            pltpu.make_async_copy(
                out_buf.at[buf, c],
                out_ref.at[pl.ds((i - 2) * BN + c * BC, BC), :],
                out_sem.at[buf, c]).wait()

        rows = pl.ds(c * BC, BC)
        x1 = x1_ref[rows, :]
        x2 = x2_ref[rows, :]
        logit1 = (jnp.dot(x1, w1a, preferred_element_type=jnp.float32)
                  + jnp.dot(x2, w1b, preferred_element_type=jnp.float32)
                  + b1)
        logit2 = (jnp.dot(x1, w2a, preferred_element_type=jnp.float32)
                  + jnp.dot(x2, w2b, preferred_element_type=jnp.float32)
                  + b2)
        g1 = 0.5 * jnp.tanh(0.5 * logit1) + 0.5
        g2 = 0.5 * jnp.tanh(0.5 * logit2) + 0.5
        out_buf[buf, c] = g1 * x1 + g2 * x2
        pltpu.make_async_copy(
            out_buf.at[buf, c],
            out_ref.at[pl.ds(i * BN + c * BC, BC), :],
            out_sem.at[buf, c]).start()

    @pl.when(i == nsteps - 1)
    def _drain():
        for c in range(NCHUNK):
            pltpu.make_async_copy(
                out_buf.at[1 - buf, c],
                out_ref.at[pl.ds((i - 1) * BN + c * BC, BC), :],
                out_sem.at[1 - buf, c]).wait()
            pltpu.make_async_copy(
                out_buf.at[buf, c],
                out_ref.at[pl.ds(i * BN + c * BC, BC), :],
                out_sem.at[buf, c]).wait()


@jax.jit
def _fused_gate(h_coa, h_aoa, W1, b1, W2, b2):
    n = h_coa.shape[0]
    grid = (n // BN,)
    row_block = pl.BlockSpec((BN, D), lambda i: (i, 0))
    full = pl.BlockSpec((D, D), lambda i: (0, 0))
    bias = pl.BlockSpec((1, D), lambda i: (0, 0))
    return pl.pallas_call(
        _fused_gate_kernel,
        grid=grid,
        in_specs=[row_block, row_block, full, full, bias, full, full, bias],
        out_specs=pl.BlockSpec(memory_space=pl.ANY),
        out_shape=jax.ShapeDtypeStruct((n, D), jnp.float32),
        scratch_shapes=[
            pltpu.VMEM((2, NCHUNK, BC, D), jnp.float32),
            pltpu.SemaphoreType.DMA((2, NCHUNK)),
        ],
        compiler_params=pltpu.CompilerParams(
            dimension_semantics=("arbitrary",)),
    )(h_coa, h_aoa, W1[:D], W1[D:], b1.reshape(1, D), W2[:D], W2[D:],
      b2.reshape(1, D))


def kernel(h_coa, h_aoa, W1, b1, W2, b2):
    return _fused_gate(h_coa, h_aoa, W1, b1, W2, b2)


# speedup vs baseline: 1.0439x; 1.0439x over previous
import jax
import jax.numpy as jnp
from jax.experimental import pallas as pl
from jax.experimental.pallas import tpu as pltpu

N = 100000
D = 128
BN = 10000
BC = 5000
NCHUNK = BN // BC


def _fused_gate_kernel(x1_ref, x2_ref, w1a_ref, w1b_ref, b1_ref,
                       w2a_ref, w2b_ref, b2_ref, out_ref,
                       out_buf, out_sem):
    i = pl.program_id(0)
    nsteps = pl.num_programs(0)
    buf = jax.lax.rem(i, 2)
    w1a = w1a_ref[...]
    w1b = w1b_ref[...]
    w2a = w2a_ref[...]
    w2b = w2b_ref[...]
    b1 = b1_ref[...]
    b2 = b2_ref[...]
    for c in range(NCHUNK):
        @pl.when(i >= 2)
        def _wait_prev():
            pltpu.make_async_copy(
                out_buf.at[buf, c],
                out_ref.at[pl.ds((i - 2) * BN + c * BC, BC), :],
                out_sem.at[buf, c]).wait()

        rows = pl.ds(c * BC, BC)
        x1 = x1_ref[rows, :]
        x2 = x2_ref[rows, :]
        logit1 = (jnp.dot(x1, w1a, preferred_element_type=jnp.float32)
                  + jnp.dot(x2, w1b, preferred_element_type=jnp.float32)
                  + b1)
        logit2 = (jnp.dot(x1, w2a, preferred_element_type=jnp.float32)
                  + jnp.dot(x2, w2b, preferred_element_type=jnp.float32)
                  + b2)
        g1 = 0.5 * jnp.tanh(0.5 * logit1) + 0.5
        g2 = 0.5 * jnp.tanh(0.5 * logit2) + 0.5
        out_buf[buf, c] = g1 * x1 + g2 * x2
        pltpu.make_async_copy(
            out_buf.at[buf, c],
            out_ref.at[pl.ds(i * BN + c * BC, BC), :],
            out_sem.at[buf, c]).start()

    @pl.when(i == nsteps - 1)
    def _drain():
        for c in range(NCHUNK):
            pltpu.make_async_copy(
                out_buf.at[1 - buf, c],
                out_ref.at[pl.ds((i - 1) * BN + c * BC, BC), :],
                out_sem.at[1 - buf, c]).wait()
            pltpu.make_async_copy(
                out_buf.at[buf, c],
                out_ref.at[pl.ds(i * BN + c * BC, BC), :],
                out_sem.at[buf, c]).wait()


@jax.jit
def _fused_gate(h_coa, h_aoa, W1, b1, W2, b2):
    n = h_coa.shape[0]
    grid = (n // BN,)
    row_block = pl.BlockSpec((BN, D), lambda i: (i, 0))
    full = pl.BlockSpec((D, D), lambda i: (0, 0))
    bias = pl.BlockSpec((1, D), lambda i: (0, 0))
    return pl.pallas_call(
        _fused_gate_kernel,
        grid=grid,
        in_specs=[row_block, row_block, full, full, bias, full, full, bias],
        out_specs=pl.BlockSpec(memory_space=pl.ANY),
        out_shape=jax.ShapeDtypeStruct((n, D), jnp.float32),
        scratch_shapes=[
            pltpu.VMEM((2, NCHUNK, BC, D), jnp.float32),
            pltpu.SemaphoreType.DMA((2, NCHUNK)),
        ],
        compiler_params=pltpu.CompilerParams(
            dimension_semantics=("arbitrary",)),
    )(h_coa, h_aoa, W1[:D], W1[D:], b1.reshape(1, D), W2[:D], W2[D:],
      b2.reshape(1, D))


def kernel(h_coa, h_aoa, W1, b1, W2, b2):
    return _fused_gate(h_coa, h_aoa, W1, b1, W2, b2)


# speedup vs baseline: 1.1883x; 1.1383x over previous
import jax
import jax.numpy as jnp
from jax.experimental import pallas as pl
from jax.experimental.pallas import tpu as pltpu

N = 100000
D = 128
BN = 10000


def _fused_gate_kernel(x1_ref, x2_ref, w1a_ref, w1b_ref, b1_ref,
                       w2a_ref, w2b_ref, b2_ref, out_ref):
    x1 = x1_ref[...]
    x2 = x2_ref[...]
    logit1 = (jnp.dot(x1, w1a_ref[...], preferred_element_type=jnp.float32)
              + jnp.dot(x2, w1b_ref[...], preferred_element_type=jnp.float32)
              + b1_ref[...])
    logit2 = (jnp.dot(x1, w2a_ref[...], preferred_element_type=jnp.float32)
              + jnp.dot(x2, w2b_ref[...], preferred_element_type=jnp.float32)
              + b2_ref[...])
    g1 = 0.5 * jnp.tanh(0.5 * logit1) + 0.5
    g2 = 0.5 * jnp.tanh(0.5 * logit2) + 0.5
    out_ref[...] = g1 * x1 + g2 * x2


@jax.jit
def _fused_gate(h_coa, h_aoa, W1, b1, W2, b2):
    n = h_coa.shape[0]
    grid = (n // BN,)
    row_block = pl.BlockSpec((BN, D), lambda i: (i, 0))
    full = pl.BlockSpec((D, D), lambda i: (0, 0))
    bias = pl.BlockSpec((1, D), lambda i: (0, 0))
    return pl.pallas_call(
        _fused_gate_kernel,
        grid=grid,
        in_specs=[row_block, row_block, full, full, bias, full, full, bias],
        out_specs=row_block,
        out_shape=jax.ShapeDtypeStruct((n, D), jnp.float32),
        compiler_params=pltpu.CompilerParams(
            dimension_semantics=("arbitrary",)),
    )(h_coa, h_aoa, W1[:D], W1[D:], b1.reshape(1, D), W2[:D], W2[D:],
      b2.reshape(1, D))


def kernel(h_coa, h_aoa, W1, b1, W2, b2):
    return _fused_gate(h_coa, h_aoa, W1, b1, W2, b2)
